# Initial kernel scaffold; baseline (speedup 1.0000x reference)
import jax
import jax.numpy as jnp
from jax.experimental import pallas as pl


def kernel(loc_data, conf_data, prior_data):
    raise NotImplementedError("write your pallas kernel here")


# speedup vs baseline: 1.8942x; 1.8942x over previous
import functools

import jax
import jax.numpy as jnp
from jax import lax
from jax.experimental import pallas as pl
from jax.experimental.pallas import tpu as pltpu
from jax.experimental.pallas import tpu_sc as plsc

B = 8
P = 20000
C = 21
NC = C - 1
LANES = B * NC
TOP_K = 200
N_CAND = 400
CONF_THRESH = 0.01
NMS_THRESH = 0.45
PPAD = 20480
PBLK = 2048
G4 = 8

_F32_INF_BITS = 0x7F800000


def _k1_body(conf_ref, loc_ref, pri_ref, sc_ref, bx_ref, by_ref, bX_ref, bY_ref):
    x = conf_ref[0]
    mx = jnp.max(x, axis=1, keepdims=True)
    e = jnp.exp(x - mx)
    prob = e / jnp.sum(e, axis=1, keepdims=True)
    s = prob[:, 1:]
    s = jnp.where(s > CONF_THRESH, s, 0.0)
    st = s.T
    gcol = pl.program_id(1) * PBLK + lax.broadcasted_iota(jnp.int32, (NC, PBLK), 1)
    sc_ref[0] = jnp.where(gcol < P, st, -1.0)

    l = loc_ref[0]
    pr = pri_ref[...]
    lx, ly, lw, lh = l[0:1], l[1:2], l[2:3], l[3:4]
    px, py, pw, ph = pr[0:1], pr[1:2], pr[2:3], pr[3:4]
    cx = px + lx * 0.1 * pw
    cy = py + ly * 0.1 * ph
    w = pw * jnp.exp(lw * 0.2)
    h = ph * jnp.exp(lh * 0.2)
    x1 = cx - w / 2.0
    y1 = cy - h / 2.0
    bx_ref[0] = x1
    by_ref[0] = y1
    bX_ref[0] = x1 + w
    bY_ref[0] = y1 + h


def _stage1(conf_p, loc_t, pri_t):
    grid = (B, PPAD // PBLK)
    return pl.pallas_call(
        _k1_body,
        grid=grid,
        in_specs=[
            pl.BlockSpec((1, PBLK, C), lambda b, j: (b, j, 0)),
            pl.BlockSpec((1, 4, PBLK), lambda b, j: (b, 0, j)),
            pl.BlockSpec((4, PBLK), lambda b, j: (0, j)),
        ],
        out_specs=[
            pl.BlockSpec((1, NC, PBLK), lambda b, j: (b, 0, j)),
            pl.BlockSpec((1, 1, PBLK), lambda b, j: (b, 0, j)),
            pl.BlockSpec((1, 1, PBLK), lambda b, j: (b, 0, j)),
            pl.BlockSpec((1, 1, PBLK), lambda b, j: (b, 0, j)),
            pl.BlockSpec((1, 1, PBLK), lambda b, j: (b, 0, j)),
        ],
        out_shape=[
            jax.ShapeDtypeStruct((B, NC, PPAD), jnp.float32),
            jax.ShapeDtypeStruct((B, 1, PPAD), jnp.float32),
            jax.ShapeDtypeStruct((B, 1, PPAD), jnp.float32),
            jax.ShapeDtypeStruct((B, 1, PPAD), jnp.float32),
            jax.ShapeDtypeStruct((B, 1, PPAD), jnp.float32),
        ],
    )(conf_p, loc_t, pri_t)


def _k2_body(sc_ref, t_ref):
    bits = lax.bitcast_convert_type(sc_ref[0], jnp.int32)

    def it(_, carry):
        lo, hi = carry
        mid = lo + ((hi - lo + 1) >> 1)
        cnt = jnp.sum((bits >= mid).astype(jnp.int32), axis=1, keepdims=True)
        ge = cnt >= N_CAND
        return jnp.where(ge, mid, lo), jnp.where(ge, hi, mid - 1)

    lo = jnp.zeros((NC, 1), jnp.int32)
    hi = jnp.full((NC, 1), _F32_INF_BITS, jnp.int32)
    lo, hi = lax.fori_loop(0, 31, it, (lo, hi))
    t_ref[0] = jnp.broadcast_to(lo, (NC, 128))


def _stage2(scores):
    return pl.pallas_call(
        _k2_body,
        grid=(B,),
        in_specs=[pl.BlockSpec((1, NC, PPAD), lambda b: (b, 0, 0))],
        out_specs=pl.BlockSpec((1, NC, 128), lambda b: (b, 0, 0)),
        out_shape=jax.ShapeDtypeStruct((B, NC, 128), jnp.int32),
    )(scores)


def _k3_body(sc_hbm, t_hbm, bx_hbm, by_hbm, bX_hbm, bY_hbm,
             cv_hbm, ci_hbm, ox_hbm, oy_hbm, oX_hbm, oY_hbm,
             sv, tv, av, ai, bv, bi, mv, mi, gi, g0, g1, g2, g3, sem):
    wid = lax.axis_index("s") * 2 + lax.axis_index("c")

    for kk in range(32):
        gi[pl.ds(kk * 16, 16)] = jnp.zeros((16,), jnp.int32)

    for t in range(5):
        lane = wid * 5 + t
        pltpu.sync_copy(sc_hbm.at[pl.ds(lane * PPAD, PPAD)], sv)
        pltpu.sync_copy(t_hbm.at[pl.ds(lane * 128, 16)], tv)
        T = tv[...]

        def step(j, carry):
            ca, cb = carry
            v = sv[pl.ds(j * 16, 16)]
            mge = v >= T
            n = jnp.sum(mge.astype(jnp.int32))

            def slow(op):
                ca, cb = op
                iv = lax.iota(jnp.int32, 16) + j * 16
                mgt = v > T
                meq = mge & (~mgt)
                igt = mgt.astype(jnp.int32)
                ieq = meq.astype(jnp.int32)
                exgt = plsc.cumsum(igt) - igt
                exeq = plsc.cumsum(ieq) - ieq
                idx_a = ca + exgt
                plsc.store_scatter(av, [idx_a], v, mask=mgt)
                plsc.store_scatter(ai, [idx_a], iv, mask=mgt)
                tie = cb + exeq
                m_b = meq & (tie < N_CAND)
                plsc.store_scatter(bv, [tie], v, mask=m_b)
                plsc.store_scatter(bi, [tie], iv, mask=m_b)
                return ca + jnp.sum(igt), cb + jnp.sum(ieq)

            return lax.cond(n > 0, slow, lambda op: op, (ca, cb))

        m, _ = lax.fori_loop(0, PPAD // 16, step, (jnp.int32(0), jnp.int32(0)))

        boff = (lane // 20) * PPAD

        def mstep(k, _):
            pos = lax.iota(jnp.int32, 16) + k * 16
            use_a = pos < m
            avv = plsc.load_gather(av, [pos])
            aii = plsc.load_gather(ai, [pos])
            pb = jnp.maximum(pos - m, 0)
            bvv = plsc.load_gather(bv, [pb])
            bii = plsc.load_gather(bi, [pb])
            val = jnp.where(use_a, avv, bvv)
            ind = jnp.where(use_a, aii, bii)
            mv[pl.ds(k * 16, 16)] = val
            mi[pl.ds(k * 16, 16)] = ind
            gi[pl.ds(k * 16, 16)] = ind + boff
            return 0

        lax.fori_loop(0, N_CAND // 16, mstep, 0)

        pltpu.sync_copy(mv, cv_hbm.at[pl.ds(lane * N_CAND, N_CAND)])
        pltpu.sync_copy(mi, ci_hbm.at[pl.ds(lane * N_CAND, N_CAND)])

        descs = []
        for tab, dst in ((bx_hbm, g0), (by_hbm, g1), (bX_hbm, g2), (bY_hbm, g3)):
            for r in range(4):
                descs.append(pltpu.async_copy(
                    tab.at[gi.at[pl.ds(r * 128, 128)]],
                    dst.at[pl.ds(r * 128, 128)], sem))
        for d in descs:
            d.wait()
        for dst, out in ((g0, ox_hbm), (g1, oy_hbm), (g2, oX_hbm), (g3, oY_hbm)):
            pltpu.sync_copy(dst.at[pl.ds(0, N_CAND)], out.at[pl.ds(lane * N_CAND, N_CAND)])


def _stage3(scores1d, t1d, bx1d, by1d, bX1d, bY1d):
    n_out = LANES * N_CAND
    f32 = jnp.float32
    run = pl.kernel(
        _k3_body,
        out_type=[
            jax.ShapeDtypeStruct((n_out,), f32),
            jax.ShapeDtypeStruct((n_out,), jnp.int32),
            jax.ShapeDtypeStruct((n_out,), f32),
            jax.ShapeDtypeStruct((n_out,), f32),
            jax.ShapeDtypeStruct((n_out,), f32),
            jax.ShapeDtypeStruct((n_out,), f32),
        ],
        mesh=plsc.VectorSubcoreMesh(core_axis_name="c", subcore_axis_name="s"),
        compiler_params=pltpu.CompilerParams(needs_layout_passes=False),
        scratch_types=[
            pltpu.VMEM((PPAD,), f32),
            pltpu.VMEM((16,), f32),
            pltpu.VMEM((416,), f32),
            pltpu.VMEM((416,), jnp.int32),
            pltpu.VMEM((416,), f32),
            pltpu.VMEM((416,), jnp.int32),
            pltpu.VMEM((N_CAND,), f32),
            pltpu.VMEM((N_CAND,), jnp.int32),
            pltpu.VMEM((512,), jnp.int32),
            pltpu.VMEM((512,), f32),
            pltpu.VMEM((512,), f32),
            pltpu.VMEM((512,), f32),
            pltpu.VMEM((512,), f32),
            pltpu.SemaphoreType.DMA,
        ],
    )
    return run(scores1d, t1d, bx1d, by1d, bX1d, bY1d)


def _k4_body(cv_ref, ci_ref, x1_ref, y1_ref, x2_ref, y2_ref, out_ref, s_scr):
    f32 = jnp.float32
    vals = cv_ref[0]
    idx = ci_ref[0]

    vj = vals[:, None, :]
    vi = vals[:, :, None]
    ij = idx[:, None, :]
    ii = idx[:, :, None]
    before = (vj > vi) | ((vj == vi) & (ij < ii))
    rank = jnp.sum(before.astype(f32), axis=2)
    kio = lax.broadcasted_iota(jnp.int32, (G4, N_CAND, N_CAND), 2)
    perm = (rank[:, :, None].astype(jnp.int32) == kio).astype(f32)

    data = jnp.stack(
        [vals, x1_ref[0], y1_ref[0], x2_ref[0], y2_ref[0]], axis=2)
    srt = lax.dot_general(perm, data, (((1,), (1,)), ((0,), (0,))),
                          precision=lax.Precision.HIGHEST,
                          preferred_element_type=f32)
    sv = srt[:, :, 0]
    x1 = srt[:, :, 1]
    y1 = srt[:, :, 2]
    x2 = srt[:, :, 3]
    y2 = srt[:, :, 4]

    area = (x2 - x1) * (y2 - y1)
    xx1 = jnp.maximum(x1[:, :, None], x1[:, None, :])
    yy1 = jnp.maximum(y1[:, :, None], y1[:, None, :])
    xx2 = jnp.minimum(x2[:, :, None], x2[:, None, :])
    yy2 = jnp.minimum(y2[:, :, None], y2[:, None, :])
    w = jnp.clip(xx2 - xx1, 0.0)
    h = jnp.clip(yy2 - yy1, 0.0)
    inter = w * h
    union = area[:, :, None] + area[:, None, :] - inter
    iou = inter / jnp.maximum(union, 1e-9)

    i_io = lax.broadcasted_iota(jnp.int32, (G4, N_CAND, N_CAND), 1)
    j_io = lax.broadcasted_iota(jnp.int32, (G4, N_CAND, N_CAND), 2)
    s_scr[...] = ((iou > NMS_THRESH) & (j_io > i_io)).astype(f32)

    keep = jnp.ones((G4, N_CAND), f32)
    for blk in range(N_CAND // 8):
        chunk = s_scr[:, blk * 8:(blk + 1) * 8, :]
        for r in range(8):
            i = blk * 8 + r
            row = chunk[:, r, :]
            ki = keep[:, i:i + 1]
            keep = keep * (1.0 - ki * row)

    ks = sv * keep

    pos = (ks > 0.0).astype(f32)
    a_io = lax.broadcasted_iota(jnp.int32, (N_CAND, N_CAND), 0)
    b_io = lax.broadcasted_iota(jnp.int32, (N_CAND, N_CAND), 1)
    ltri = (a_io < b_io).astype(f32)
    cume = lax.dot_general(pos, ltri, (((1,), (0,)), ((), ())),
                           precision=lax.Precision.HIGHEST,
                           preferred_element_type=f32)
    ptot = jnp.sum(pos, axis=1, keepdims=True)
    row_io = lax.broadcasted_iota(jnp.int32, (G4, N_CAND), 1).astype(f32)
    rank2 = jnp.where(ks > 0.0, cume, ptot + row_io - cume).astype(jnp.int32)
    kio2t = lax.broadcasted_iota(jnp.int32, (G4, TOP_K, N_CAND), 1)
    perm2t = (rank2[:, None, :] == kio2t).astype(f32)
    data2 = jnp.stack([ks, x1, y1, x2, y2], axis=2)
    out_ref[0] = lax.dot_general(perm2t, data2, (((2,), (1,)), ((0,), (0,))),
                                 precision=lax.Precision.HIGHEST,
                                 preferred_element_type=f32)


def _stage4(cv, ci, bx, by, bX, bY):
    ng = LANES // G4
    args = [a.reshape(ng, G4, N_CAND) for a in (cv, ci, bx, by, bX, bY)]
    out = pl.pallas_call(
        _k4_body,
        grid=(ng,),
        in_specs=[pl.BlockSpec((1, G4, N_CAND), lambda g: (g, 0, 0))] * 6,
        out_specs=pl.BlockSpec((1, G4, TOP_K, 5), lambda g: (g, 0, 0, 0)),
        out_shape=jax.ShapeDtypeStruct((ng, G4, TOP_K, 5), jnp.float32),
        scratch_shapes=[pltpu.VMEM((G4, N_CAND, N_CAND), jnp.float32)],
    )(*args)
    return out.reshape(LANES, TOP_K, 5)


@jax.jit
def kernel(loc_data, conf_data, prior_data):
    conf_p = jnp.pad(conf_data, ((0, 0), (0, PPAD - P), (0, 0)))
    loc_t = jnp.pad(loc_data, ((0, 0), (0, PPAD - P), (0, 0))).transpose(0, 2, 1)
    pri_t = jnp.pad(prior_data, ((0, PPAD - P), (0, 0))).T

    scores, bx, by, bX, bY = _stage1(conf_p, loc_t, pri_t)
    tbits = _stage2(scores)
    tflt = lax.bitcast_convert_type(tbits, jnp.float32)
    cv, ci, cbx, cby, cbX, cbY = _stage3(
        scores.reshape(-1), tflt.reshape(-1),
        bx.reshape(-1), by.reshape(-1), bX.reshape(-1), bY.reshape(-1))
    sh = (LANES, N_CAND)
    out = _stage4(cv.reshape(sh), ci.reshape(sh), cbx.reshape(sh),
                  cby.reshape(sh), cbX.reshape(sh), cbY.reshape(sh))
    return out.reshape(B, NC, TOP_K, 5)
